# baseline (device time: 33602 ns/iter reference)
import jax
import jax.numpy as jnp
from jax.experimental import pallas as pl
from jax.experimental.pallas import tpu as pltpu

BLK = 1024


def kernel(x):
    m, n = x.shape

    def body(x_ref, out_ref, carry_ref):
        i = pl.program_id(0)

        @pl.when(i == 0)
        def _():
            carry_ref[...] = jnp.ones((1, n), jnp.float32)

        one = lambda *shape: jnp.ones(shape, jnp.float32)
        r = x_ref[...].reshape(128, 8, n)
        for s in (1, 2, 4):
            r = r * jnp.concatenate([one(128, s, n), r[:, :8 - s, :]], axis=1)
        t = r[:, 7:8, :].reshape(16, 8, n)
        for s in (1, 2, 4):
            t = t * jnp.concatenate([one(16, s, n), t[:, :8 - s, :]], axis=1)
        u = t[:, 7:8, :]
        for s in (1, 2, 4, 8):
            u = u * jnp.concatenate([one(s, 1, n), u[:16 - s, :, :]], axis=0)
        exc_u = jnp.concatenate([one(1, 1, n), u[:15]], axis=0)
        exc_t = jnp.concatenate([one(16, 1, n), t[:, :7, :]], axis=1)
        scale = (exc_t * exc_u).reshape(128, 1, n)
        y = (r * (scale * carry_ref[...].reshape(1, 1, n))).reshape(BLK, n)
        out_ref[...] = y
        carry_ref[...] = y[BLK - 1:BLK, :]

    return pl.pallas_call(
        body,
        grid=(m // BLK,),
        out_shape=jax.ShapeDtypeStruct((m, n), jnp.float32),
        in_specs=[pl.BlockSpec((BLK, n), lambda i: (i, 0))],
        out_specs=pl.BlockSpec((BLK, n), lambda i: (i, 0)),
        scratch_shapes=[pltpu.VMEM((1, n), jnp.float32)],
        compiler_params=pltpu.CompilerParams(
            dimension_semantics=("arbitrary",),
            vmem_limit_bytes=60 * 1024 * 1024,
        ),
    )(x)
